# Initial kernel scaffold; baseline (speedup 1.0000x reference)
import jax
import jax.numpy as jnp
from jax.experimental import pallas as pl


def kernel(x, edge_index, W1, b1, W2, b2):
    raise NotImplementedError("write your pallas kernel here")


# speedup vs baseline: 6.8528x; 6.8528x over previous
import functools

import jax
import jax.numpy as jnp
from jax import lax
from jax.experimental import pallas as pl
from jax.experimental.pallas import tpu as pltpu
from jax.experimental.pallas import tpu_sc as plsc

N = 10000
E = 320000
D = 128

N_PAD = 10240
E_PAD = 327680
NC = 2
NS = 16
NW = NC * NS
EPW = E_PAD // NW
CHUNK = 128
NCHUNK = EPW // CHUNK
ROWS_PER_SUB = N_PAD // NS
DEG_W = 16

_MESH = plsc.VectorSubcoreMesh(core_axis_name="c", subcore_axis_name="s")


def _zero_vmem_2d(buf, rows, cols):
    def row_body(i, _):
        def col_body(j, _):
            buf[i, pl.ds(j * 16, 16)] = jnp.zeros((16,), jnp.float32)
            return 0
        return lax.fori_loop(0, cols // 16, col_body, 0)
    lax.fori_loop(0, rows, row_body, 0)


@functools.partial(
    pl.kernel,
    out_type=jax.ShapeDtypeStruct((NC, N_PAD, DEG_W), jnp.float32),
    mesh=_MESH,
    scratch_types=[
        pltpu.VMEM((CHUNK,), jnp.int32),
        pltpu.VMEM((CHUNK, DEG_W), jnp.float32),
        pltpu.VMEM_SHARED((N_PAD, DEG_W), jnp.float32),
    ],
)
def _sc_degree(dst_hbm, out_hbm, didx, ones, acc):
    c = lax.axis_index("c")
    s = lax.axis_index("s")
    wid = s * NC + c
    base = wid * EPW

    _zero_vmem_2d(ones, CHUNK, DEG_W)
    for j in range(ROWS_PER_SUB // CHUNK):
        pltpu.sync_copy(ones, acc.at[pl.ds(s * ROWS_PER_SUB + j * CHUNK, CHUNK)])

    def ones_body(i, _):
        ones[i, :] = jnp.ones((DEG_W,), jnp.float32)
        return 0
    lax.fori_loop(0, CHUNK, ones_body, 0)
    plsc.subcore_barrier()

    def body(i, _):
        pltpu.sync_copy(dst_hbm.at[pl.ds(base + i * CHUNK, CHUNK)], didx)
        pltpu.sync_copy(ones, acc.at[didx], add=True)
        return 0
    lax.fori_loop(0, NCHUNK, body, 0)
    plsc.subcore_barrier()

    row0 = s * ROWS_PER_SUB
    pltpu.sync_copy(acc.at[pl.ds(row0, ROWS_PER_SUB)],
                    out_hbm.at[c, pl.ds(row0, ROWS_PER_SUB)])


@functools.partial(
    pl.kernel,
    out_type=jax.ShapeDtypeStruct((NC, N_PAD, D), jnp.float32),
    mesh=_MESH,
    scratch_types=[
        pltpu.VMEM((CHUNK,), jnp.int32),
        pltpu.VMEM((CHUNK,), jnp.int32),
        pltpu.VMEM((CHUNK, D), jnp.float32),
        pltpu.VMEM_SHARED((N_PAD, D), jnp.float32),
        pltpu.SemaphoreType.DMA,
    ],
)
def _sc_segsum(g_hbm, src_hbm, dst_hbm, out_hbm, sidx, didx, rows, acc, sem):
    c = lax.axis_index("c")
    s = lax.axis_index("s")
    wid = s * NC + c
    base = wid * EPW

    _zero_vmem_2d(rows, CHUNK, D)
    for j in range(ROWS_PER_SUB // CHUNK):
        pltpu.sync_copy(rows, acc.at[pl.ds(s * ROWS_PER_SUB + j * CHUNK, CHUNK)])
    plsc.subcore_barrier()

    def body(i, _):
        off = base + i * CHUNK
        pltpu.sync_copy(src_hbm.at[pl.ds(off, CHUNK)], sidx)
        pltpu.sync_copy(dst_hbm.at[pl.ds(off, CHUNK)], didx)
        pltpu.async_copy(g_hbm.at[sidx], rows, sem).wait()
        pltpu.sync_copy(rows, acc.at[didx], add=True)
        return 0
    lax.fori_loop(0, NCHUNK, body, 0)
    plsc.subcore_barrier()

    row0 = s * ROWS_PER_SUB
    pltpu.sync_copy(acc.at[pl.ds(row0, ROWS_PER_SUB)],
                    out_hbm.at[c, pl.ds(row0, ROWS_PER_SUB)])


ROW_BLK = 512
GRID = N_PAD // ROW_BLK


def _dinv_block(degp_ref):
    deg = degp_ref[0, :, 0:1] + degp_ref[1, :, 0:1] + 1.0
    return lax.rsqrt(deg)


def _tc_pre_body(x_ref, w_ref, degp_ref, g_ref):
    h = jnp.dot(x_ref[...], w_ref[...], preferred_element_type=jnp.float32)
    g_ref[...] = h * _dinv_block(degp_ref)


def _tc_mid_body(p_ref, g_ref, degp_ref, w_ref, b_ref, o_ref):
    dinv = _dinv_block(degp_ref)
    s = p_ref[0] + p_ref[1] + g_ref[...]
    h1 = jnp.maximum(s * dinv + b_ref[...], 0.0)
    o_ref[...] = jnp.dot(h1, w_ref[...], preferred_element_type=jnp.float32) * dinv


def _tc_post_body(p_ref, g_ref, degp_ref, b_ref, o_ref):
    dinv = _dinv_block(degp_ref)
    s = p_ref[0] + p_ref[1] + g_ref[...]
    o_ref[...] = s * dinv + b_ref[...]


_ROWS_SPEC = pl.BlockSpec((ROW_BLK, D), lambda i: (i, 0))
_PARTS_SPEC = pl.BlockSpec((NC, ROW_BLK, D), lambda i: (0, i, 0))
_DEGP_SPEC = pl.BlockSpec((NC, ROW_BLK, DEG_W), lambda i: (0, i, 0))
_W_SPEC = pl.BlockSpec((D, D), lambda i: (0, 0))
_B_SPEC = pl.BlockSpec((1, D), lambda i: (0, 0))
_OUT_SHAPE = jax.ShapeDtypeStruct((N_PAD, D), jnp.float32)

_tc_pre = pl.pallas_call(
    _tc_pre_body, grid=(GRID,),
    in_specs=[_ROWS_SPEC, _W_SPEC, _DEGP_SPEC],
    out_specs=_ROWS_SPEC, out_shape=_OUT_SHAPE)

_tc_mid = pl.pallas_call(
    _tc_mid_body, grid=(GRID,),
    in_specs=[_PARTS_SPEC, _ROWS_SPEC, _DEGP_SPEC, _W_SPEC, _B_SPEC],
    out_specs=_ROWS_SPEC, out_shape=_OUT_SHAPE)

_tc_post = pl.pallas_call(
    _tc_post_body, grid=(GRID,),
    in_specs=[_PARTS_SPEC, _ROWS_SPEC, _DEGP_SPEC, _B_SPEC],
    out_specs=_ROWS_SPEC, out_shape=_OUT_SHAPE)


@jax.jit
def kernel(x, edge_index, W1, b1, W2, b2):
    src = edge_index[0].astype(jnp.int32)
    dst = edge_index[1].astype(jnp.int32)
    pad = jnp.full((E_PAD - E,), N, dtype=jnp.int32)
    srcp = jnp.concatenate([src, pad])
    dstp = jnp.concatenate([dst, pad])
    xp = jnp.pad(x, ((0, N_PAD - N), (0, 0)))
    b1r = b1.reshape(1, D)
    b2r = b2.reshape(1, D)

    degp = _sc_degree(dstp)
    g1 = _tc_pre(xp, W1, degp)
    p1 = _sc_segsum(g1, srcp, dstp)
    g2 = _tc_mid(p1, g1, degp, W2, b1r)
    p2 = _sc_segsum(g2, srcp, dstp)
    out = _tc_post(p2, g2, degp, b2r)
    return out[:N]


# speedup vs baseline: 8.2333x; 1.2014x over previous
import functools

import jax
import jax.numpy as jnp
from jax import lax
from jax.experimental import pallas as pl
from jax.experimental.pallas import tpu as pltpu
from jax.experimental.pallas import tpu_sc as plsc

N = 10000
E = 320000
D = 128

N_PAD = 10240
E_PAD = 327680
NC = 2
NS = 16
NW = NC * NS
EPW = E_PAD // NW
CHUNK = 128
NCHUNK = EPW // CHUNK
NBUF = 2
SCHUNK = 16
NSUPER = NCHUNK // SCHUNK
ROWS_PER_SUB = N_PAD // NS
DEG_W = 16

_MESH = plsc.VectorSubcoreMesh(core_axis_name="c", subcore_axis_name="s")


def _zero_vmem_2d(buf, rows, cols):
    def row_body(i, _):
        def col_body(j, _):
            buf[i, pl.ds(j * 16, 16)] = jnp.zeros((16,), jnp.float32)
            return 0
        return lax.fori_loop(0, cols // 16, col_body, 0)
    lax.fori_loop(0, rows, row_body, 0)


@functools.partial(
    pl.kernel,
    out_type=jax.ShapeDtypeStruct((NC, N_PAD, DEG_W), jnp.float32),
    mesh=_MESH,
    scratch_types=[
        pltpu.VMEM((SCHUNK, CHUNK), jnp.int32),
        pltpu.VMEM((CHUNK, DEG_W), jnp.float32),
        pltpu.VMEM_SHARED((N_PAD, DEG_W), jnp.float32),
        pltpu.SemaphoreType.DMA,
    ],
)
def _sc_degree(dst_hbm, out_hbm, didx, ones, acc, sem):
    c = lax.axis_index("c")
    s = lax.axis_index("s")
    wid = s * NC + c

    _zero_vmem_2d(ones, CHUNK, DEG_W)
    for j in range(ROWS_PER_SUB // CHUNK):
        pltpu.sync_copy(ones, acc.at[pl.ds(s * ROWS_PER_SUB + j * CHUNK, CHUNK)])

    def ones_body(i, _):
        ones[i, :] = jnp.ones((DEG_W,), jnp.float32)
        return 0
    lax.fori_loop(0, CHUNK, ones_body, 0)
    plsc.subcore_barrier()

    def super_body(u, _):
        pltpu.sync_copy(dst_hbm.at[wid, u], didx)

        def body(i, _):
            pltpu.sync_copy(ones, acc.at[didx.at[i]], add=True)
            return 0
        lax.fori_loop(0, SCHUNK, body, 0)
        return 0
    lax.fori_loop(0, NSUPER, super_body, 0)
    plsc.subcore_barrier()

    row0 = s * ROWS_PER_SUB
    pltpu.sync_copy(acc.at[pl.ds(row0, ROWS_PER_SUB)],
                    out_hbm.at[c, pl.ds(row0, ROWS_PER_SUB)])


@functools.partial(
    pl.kernel,
    out_type=jax.ShapeDtypeStruct((NC, N_PAD, D), jnp.float32),
    mesh=_MESH,
    scratch_types=[
        pltpu.VMEM((SCHUNK, CHUNK), jnp.int32),
        pltpu.VMEM((SCHUNK, CHUNK), jnp.int32),
        pltpu.VMEM((CHUNK, D), jnp.float32),
        pltpu.VMEM((CHUNK, D), jnp.float32),
        pltpu.VMEM_SHARED((N_PAD, D), jnp.float32),
        pltpu.SemaphoreType.DMA,
        pltpu.SemaphoreType.DMA,
        pltpu.SemaphoreType.DMA,
        pltpu.SemaphoreType.DMA,
    ],
)
def _sc_segsum(g_hbm, src_hbm, dst_hbm, out_hbm, sidx, didx, buf0, buf1, acc,
               gsem0, gsem1, ssem0, ssem1):
    c = lax.axis_index("c")
    s = lax.axis_index("s")
    wid = s * NC + c
    bufs = [buf0, buf1]
    gsems = [gsem0, gsem1]
    ssems = [ssem0, ssem1]

    _zero_vmem_2d(buf0, CHUNK, D)
    for j in range(ROWS_PER_SUB // CHUNK):
        pltpu.sync_copy(buf0, acc.at[pl.ds(s * ROWS_PER_SUB + j * CHUNK, CHUNK)])
    plsc.subcore_barrier()

    def gather_start(b, i):
        pltpu.async_copy(g_hbm.at[sidx.at[i]], bufs[b], gsems[b])

    def gather_wait(b):
        pltpu.make_async_copy(g_hbm.at[sidx.at[0]], bufs[b], gsems[b]).wait()

    def scatter_start(b, i):
        return pltpu.async_copy(bufs[b], acc.at[didx.at[i]], ssems[b], add=True)

    def super_body(u, _):
        pltpu.sync_copy(src_hbm.at[wid, u], sidx)
        pltpu.sync_copy(dst_hbm.at[wid, u], didx)
        for b in range(NBUF):
            gather_start(b, b)

        def inner(g, _):
            descs = []
            for b in range(NBUF):
                gather_wait(b)
                descs.append(scatter_start(b, g * NBUF + b))
            for b in range(NBUF):
                descs[b].wait()
                gather_start(b, g * NBUF + b + NBUF)
            return 0
        lax.fori_loop(0, SCHUNK // NBUF - 1, inner, 0)
        descs = []
        for b in range(NBUF):
            gather_wait(b)
            descs.append(scatter_start(b, SCHUNK - NBUF + b))
        for d in descs:
            d.wait()
        return 0
    lax.fori_loop(0, NSUPER, super_body, 0)
    plsc.subcore_barrier()

    row0 = s * ROWS_PER_SUB
    pltpu.sync_copy(acc.at[pl.ds(row0, ROWS_PER_SUB)],
                    out_hbm.at[c, pl.ds(row0, ROWS_PER_SUB)])


ROW_BLK = 512
GRID = N_PAD // ROW_BLK


def _dinv_block(degp_ref):
    deg = degp_ref[0, :, 0:1] + degp_ref[1, :, 0:1] + 1.0
    return lax.rsqrt(deg)


def _tc_pre_body(x_ref, w_ref, degp_ref, g_ref):
    h = jnp.dot(x_ref[...], w_ref[...], preferred_element_type=jnp.float32)
    g_ref[...] = h * _dinv_block(degp_ref)


def _tc_mid_body(p_ref, g_ref, degp_ref, w_ref, b_ref, o_ref):
    dinv = _dinv_block(degp_ref)
    s = p_ref[0] + p_ref[1] + g_ref[...]
    h1 = jnp.maximum(s * dinv + b_ref[...], 0.0)
    o_ref[...] = jnp.dot(h1, w_ref[...], preferred_element_type=jnp.float32) * dinv


def _tc_post_body(p_ref, g_ref, degp_ref, b_ref, o_ref):
    dinv = _dinv_block(degp_ref)
    s = p_ref[0] + p_ref[1] + g_ref[...]
    o_ref[...] = s * dinv + b_ref[...]


_ROWS_SPEC = pl.BlockSpec((ROW_BLK, D), lambda i: (i, 0))
_PARTS_SPEC = pl.BlockSpec((NC, ROW_BLK, D), lambda i: (0, i, 0))
_DEGP_SPEC = pl.BlockSpec((NC, ROW_BLK, DEG_W), lambda i: (0, i, 0))
_W_SPEC = pl.BlockSpec((D, D), lambda i: (0, 0))
_B_SPEC = pl.BlockSpec((1, D), lambda i: (0, 0))
_OUT_SHAPE = jax.ShapeDtypeStruct((N_PAD, D), jnp.float32)

_tc_pre = pl.pallas_call(
    _tc_pre_body, grid=(GRID,),
    in_specs=[_ROWS_SPEC, _W_SPEC, _DEGP_SPEC],
    out_specs=_ROWS_SPEC, out_shape=_OUT_SHAPE)

_tc_mid = pl.pallas_call(
    _tc_mid_body, grid=(GRID,),
    in_specs=[_PARTS_SPEC, _ROWS_SPEC, _DEGP_SPEC, _W_SPEC, _B_SPEC],
    out_specs=_ROWS_SPEC, out_shape=_OUT_SHAPE)

_tc_post = pl.pallas_call(
    _tc_post_body, grid=(GRID,),
    in_specs=[_PARTS_SPEC, _ROWS_SPEC, _DEGP_SPEC, _B_SPEC],
    out_specs=_ROWS_SPEC, out_shape=_OUT_SHAPE)


@jax.jit
def kernel(x, edge_index, W1, b1, W2, b2):
    src = edge_index[0].astype(jnp.int32)
    dst = edge_index[1].astype(jnp.int32)
    pad = jnp.full((E_PAD - E,), N, dtype=jnp.int32)
    srcp = jnp.concatenate([src, pad]).reshape(NW, NSUPER, SCHUNK, CHUNK)
    dstp = jnp.concatenate([dst, pad]).reshape(NW, NSUPER, SCHUNK, CHUNK)
    xp = jnp.pad(x, ((0, N_PAD - N), (0, 0)))
    b1r = b1.reshape(1, D)
    b2r = b2.reshape(1, D)

    degp = _sc_degree(dstp)
    g1 = _tc_pre(xp, W1, degp)
    p1 = _sc_segsum(g1, srcp, dstp)
    g2 = _tc_mid(p1, g1, degp, W2, b1r)
    p2 = _sc_segsum(g2, srcp, dstp)
    out = _tc_post(p2, g2, degp, b2r)
    return out[:N]


# speedup vs baseline: 9.0587x; 1.1003x over previous
import functools

import jax
import jax.numpy as jnp
from jax import lax
from jax.experimental import pallas as pl
from jax.experimental.pallas import tpu as pltpu
from jax.experimental.pallas import tpu_sc as plsc

N = 10000
E = 320000
D = 128

N_PAD = 10240
E_PAD = 327680
NC = 2
NS = 16
NW = NC * NS
EPW = E_PAD // NW
CHUNK = 80
NCHUNK = EPW // CHUNK
NBUF = 4
SCHUNK = 16
NSUPER = NCHUNK // SCHUNK
ROWS_PER_SUB = N_PAD // NS
DEG_W = 16

_MESH = plsc.VectorSubcoreMesh(core_axis_name="c", subcore_axis_name="s")


def _zero_vmem_2d(buf, rows, cols):
    def row_body(i, _):
        def col_body(j, _):
            buf[i, pl.ds(j * 16, 16)] = jnp.zeros((16,), jnp.float32)
            return 0
        return lax.fori_loop(0, cols // 16, col_body, 0)
    lax.fori_loop(0, rows, row_body, 0)


@functools.partial(
    pl.kernel,
    out_type=jax.ShapeDtypeStruct((NC, N_PAD, DEG_W), jnp.float32),
    mesh=_MESH,
    scratch_types=[
        pltpu.VMEM((SCHUNK, CHUNK), jnp.int32),
        pltpu.VMEM((CHUNK, DEG_W), jnp.float32),
        pltpu.VMEM_SHARED((N_PAD, DEG_W), jnp.float32),
        pltpu.SemaphoreType.DMA,
    ],
)
def _sc_degree(dst_hbm, out_hbm, didx, ones, acc, sem):
    c = lax.axis_index("c")
    s = lax.axis_index("s")
    wid = s * NC + c

    _zero_vmem_2d(ones, CHUNK, DEG_W)
    for j in range(ROWS_PER_SUB // CHUNK):
        pltpu.sync_copy(ones, acc.at[pl.ds(s * ROWS_PER_SUB + j * CHUNK, CHUNK)])

    def ones_body(i, _):
        ones[i, :] = jnp.ones((DEG_W,), jnp.float32)
        return 0
    lax.fori_loop(0, CHUNK, ones_body, 0)
    plsc.subcore_barrier()

    def super_body(u, _):
        pltpu.sync_copy(dst_hbm.at[wid, u], didx)

        def body(i, _):
            pltpu.sync_copy(ones, acc.at[didx.at[i]], add=True)
            return 0
        lax.fori_loop(0, SCHUNK, body, 0)
        return 0
    lax.fori_loop(0, NSUPER, super_body, 0)
    plsc.subcore_barrier()

    row0 = s * ROWS_PER_SUB
    pltpu.sync_copy(acc.at[pl.ds(row0, ROWS_PER_SUB)],
                    out_hbm.at[c, pl.ds(row0, ROWS_PER_SUB)])


@functools.partial(
    pl.kernel,
    out_type=jax.ShapeDtypeStruct((NC, N_PAD, D), jnp.float32),
    mesh=_MESH,
    scratch_types=[
        pltpu.VMEM((SCHUNK, CHUNK), jnp.int32),
        pltpu.VMEM((SCHUNK, CHUNK), jnp.int32),
        pltpu.VMEM((CHUNK, D), jnp.float32),
        pltpu.VMEM((CHUNK, D), jnp.float32),
        pltpu.VMEM((CHUNK, D), jnp.float32),
        pltpu.VMEM((CHUNK, D), jnp.float32),
        pltpu.VMEM_SHARED((N_PAD, D), jnp.float32),
        pltpu.SemaphoreType.DMA,
        pltpu.SemaphoreType.DMA,
        pltpu.SemaphoreType.DMA,
        pltpu.SemaphoreType.DMA,
        pltpu.SemaphoreType.DMA,
        pltpu.SemaphoreType.DMA,
        pltpu.SemaphoreType.DMA,
        pltpu.SemaphoreType.DMA,
    ],
)
def _sc_segsum(g_hbm, src_hbm, dst_hbm, out_hbm, sidx, didx,
               buf0, buf1, buf2, buf3, acc,
               gsem0, gsem1, gsem2, gsem3, ssem0, ssem1, ssem2, ssem3):
    c = lax.axis_index("c")
    s = lax.axis_index("s")
    wid = s * NC + c
    bufs = [buf0, buf1, buf2, buf3]
    gsems = [gsem0, gsem1, gsem2, gsem3]
    ssems = [ssem0, ssem1, ssem2, ssem3]

    _zero_vmem_2d(buf0, CHUNK, D)
    for j in range(ROWS_PER_SUB // CHUNK):
        pltpu.sync_copy(buf0, acc.at[pl.ds(s * ROWS_PER_SUB + j * CHUNK, CHUNK)])
    plsc.subcore_barrier()

    def gather_start(b, i):
        pltpu.async_copy(g_hbm.at[sidx.at[i]], bufs[b], gsems[b])

    def gather_wait(b):
        pltpu.make_async_copy(g_hbm.at[sidx.at[0]], bufs[b], gsems[b]).wait()

    def scatter_start(b, i):
        return pltpu.async_copy(bufs[b], acc.at[didx.at[i]], ssems[b], add=True)

    def super_body(u, _):
        pltpu.sync_copy(src_hbm.at[wid, u], sidx)
        pltpu.sync_copy(dst_hbm.at[wid, u], didx)
        for b in range(NBUF):
            gather_start(b, b)

        def inner(g, _):
            descs = []
            for b in range(NBUF):
                gather_wait(b)
                descs.append(scatter_start(b, g * NBUF + b))
            for b in range(NBUF):
                descs[b].wait()
                gather_start(b, g * NBUF + b + NBUF)
            return 0
        lax.fori_loop(0, SCHUNK // NBUF - 1, inner, 0)
        descs = []
        for b in range(NBUF):
            gather_wait(b)
            descs.append(scatter_start(b, SCHUNK - NBUF + b))
        for d in descs:
            d.wait()
        return 0
    lax.fori_loop(0, NSUPER, super_body, 0)
    plsc.subcore_barrier()

    row0 = s * ROWS_PER_SUB
    pltpu.sync_copy(acc.at[pl.ds(row0, ROWS_PER_SUB)],
                    out_hbm.at[c, pl.ds(row0, ROWS_PER_SUB)])


ROW_BLK = 512
GRID = N_PAD // ROW_BLK


def _dinv_block(degp_ref):
    deg = degp_ref[0, :, 0:1] + degp_ref[1, :, 0:1] + 1.0
    return lax.rsqrt(deg)


def _tc_pre_body(x_ref, w_ref, degp_ref, g_ref):
    h = jnp.dot(x_ref[...], w_ref[...], preferred_element_type=jnp.float32)
    g_ref[...] = h * _dinv_block(degp_ref)


def _tc_mid_body(p_ref, g_ref, degp_ref, w_ref, b_ref, o_ref):
    dinv = _dinv_block(degp_ref)
    s = p_ref[0] + p_ref[1] + g_ref[...]
    h1 = jnp.maximum(s * dinv + b_ref[...], 0.0)
    o_ref[...] = jnp.dot(h1, w_ref[...], preferred_element_type=jnp.float32) * dinv


def _tc_post_body(p_ref, g_ref, degp_ref, b_ref, o_ref):
    dinv = _dinv_block(degp_ref)
    s = p_ref[0] + p_ref[1] + g_ref[...]
    o_ref[...] = s * dinv + b_ref[...]


_ROWS_SPEC = pl.BlockSpec((ROW_BLK, D), lambda i: (i, 0))
_PARTS_SPEC = pl.BlockSpec((NC, ROW_BLK, D), lambda i: (0, i, 0))
_DEGP_SPEC = pl.BlockSpec((NC, ROW_BLK, DEG_W), lambda i: (0, i, 0))
_W_SPEC = pl.BlockSpec((D, D), lambda i: (0, 0))
_B_SPEC = pl.BlockSpec((1, D), lambda i: (0, 0))
_OUT_SHAPE = jax.ShapeDtypeStruct((N_PAD, D), jnp.float32)

_tc_pre = pl.pallas_call(
    _tc_pre_body, grid=(GRID,),
    in_specs=[_ROWS_SPEC, _W_SPEC, _DEGP_SPEC],
    out_specs=_ROWS_SPEC, out_shape=_OUT_SHAPE)

_tc_mid = pl.pallas_call(
    _tc_mid_body, grid=(GRID,),
    in_specs=[_PARTS_SPEC, _ROWS_SPEC, _DEGP_SPEC, _W_SPEC, _B_SPEC],
    out_specs=_ROWS_SPEC, out_shape=_OUT_SHAPE)

_tc_post = pl.pallas_call(
    _tc_post_body, grid=(GRID,),
    in_specs=[_PARTS_SPEC, _ROWS_SPEC, _DEGP_SPEC, _B_SPEC],
    out_specs=_ROWS_SPEC, out_shape=_OUT_SHAPE)


@jax.jit
def kernel(x, edge_index, W1, b1, W2, b2):
    src = edge_index[0].astype(jnp.int32)
    dst = edge_index[1].astype(jnp.int32)
    pad = jnp.full((E_PAD - E,), N, dtype=jnp.int32)
    srcp = jnp.concatenate([src, pad]).reshape(NW, NSUPER, SCHUNK, CHUNK)
    dstp = jnp.concatenate([dst, pad]).reshape(NW, NSUPER, SCHUNK, CHUNK)
    xp = jnp.pad(x, ((0, N_PAD - N), (0, 0)))
    b1r = b1.reshape(1, D)
    b2r = b2.reshape(1, D)

    degp = _sc_degree(dstp)
    g1 = _tc_pre(xp, W1, degp)
    p1 = _sc_segsum(g1, srcp, dstp)
    g2 = _tc_mid(p1, g1, degp, W2, b1r)
    p2 = _sc_segsum(g2, srcp, dstp)
    out = _tc_post(p2, g2, degp, b2r)
    return out[:N]
